# baseline (speedup 1.0000x reference)
import functools

import jax
import jax.numpy as jnp
from jax import lax
from jax.experimental import pallas as pl
from jax.experimental.pallas import tpu as pltpu
from jax.experimental.pallas import tpu_sc as plsc

B = 2
IN_C = 768
H = 128
W = 128
MAX_PROPOSALS = 30
H_BLK = 32
N_H_BLK = H // H_BLK
L = 16
C_CHUNK = 128
N_CHUNK = IN_C // C_CHUNK


def _center_topk_body(f_ref, w_ref, b_ref, s_ref, i_ref, pb_ref, center_ref):
    j = pl.program_id(0)

    x = f_ref[...].astype(jnp.bfloat16).astype(jnp.float32)
    wv = w_ref[...].astype(jnp.bfloat16).astype(jnp.float32)[None, :, :, None]
    logits = jnp.sum(x * wv, axis=1) + b_ref[0, 0]
    p = jnp.clip(1.0 / (1.0 + jnp.exp(-logits)), 0.0001, 1.0 - 0.0001)
    center_ref[pl.ds(j * H_BLK, H_BLK), :] = (p[0] + p[1]) * 0.5

    @pl.when(j == N_H_BLK - 1)
    def _():
        center = center_ref[...]

        def win3(t, pad_val, red):
            prow = jnp.full((1, W), pad_val, dtype=t.dtype)
            pcol = jnp.full((H, 1), pad_val, dtype=t.dtype)
            up = jnp.concatenate([t[1:, :], prow], axis=0)
            dn = jnp.concatenate([prow, t[:-1, :]], axis=0)
            acc = red(red(up, t), dn)
            lf = jnp.concatenate([acc[:, 1:], pcol], axis=1)
            rt = jnp.concatenate([pcol, acc[:, :-1]], axis=1)
            return red(red(lf, acc), rt)

        pool = win3(center, 0.0, jnp.add) / 9.0
        c2 = (center + pool) * 0.5
        mx = win3(c2, -jnp.inf, jnp.maximum)
        sup = jnp.where(mx == c2, c2, 0.0)

        row = jax.lax.broadcasted_iota(jnp.int32, (H, W), 0)
        col = jax.lax.broadcasted_iota(jnp.int32, (H, W), 1)
        lin = row * W + col
        out_lane = jax.lax.broadcasted_iota(jnp.int32, (1, 32), 1)
        sub8 = jax.lax.broadcasted_iota(jnp.int32, (8, W), 0)
        col8 = jax.lax.broadcasted_iota(jnp.int32, (8, W), 1)
        base8 = sub8 * W + col8

        def body(i, carry):
            v, sc, ix = carry
            r8 = v[0:8, :]
            t8 = jnp.zeros((8, W), jnp.int32)
            for t in range(1, 16):
                vt = v[8 * t:8 * t + 8, :]
                gt = vt > r8
                r8 = jnp.where(gt, vt, r8)
                t8 = jnp.where(gt, t, t8)
            m = jnp.max(r8)
            idx = jnp.min(jnp.where(r8 == m, t8 * (8 * W) + base8,
                                    jnp.int32(2**30)))
            sc = jnp.where(out_lane == i, m, sc)
            ix = jnp.where(out_lane == i, idx, ix)
            return (jnp.where(lin == idx, -jnp.inf, v), sc, ix)

        sc0 = jnp.zeros((1, 32), jnp.float32)
        ix0 = jnp.zeros((1, 32), jnp.int32)
        _, sc, ix = jax.lax.fori_loop(0, MAX_PROPOSALS, body, (sup, sc0, ix0))
        s_ref[...] = sc
        i_ref[...] = ix
        t32 = jax.lax.transpose(jnp.broadcast_to(ix, (32, 32)), (1, 0))
        pb_ref[...] = jnp.concatenate([t32, t32, t32, t32], axis=1)


def _sc_gather_body(feat_ref, posb_ref, out_ref, posrow_ref,
                    cidx0_ref, cidx1_ref, row0_ref, row1_ref, sem):
    wid = lax.axis_index("s") * 2 + lax.axis_index("c")

    @pl.when(wid < MAX_PROPOSALS)
    def _():
        lane = lax.broadcasted_iota(jnp.int32, (L,), 0)
        pltpu.sync_copy(posb_ref.at[wid], posrow_ref)
        posv = posrow_ref[pl.ds(0, L)]
        for k in range(N_CHUNK):
            for m in range(C_CHUNK // L):
                c0 = k * C_CHUNK + m * L
                cidx0_ref[k, pl.ds(m * L, L)] = posv + (c0 + lane) * (H * W)
                cidx1_ref[k, pl.ds(m * L, L)] = (IN_C * H * W) + posv + (c0 + lane) * (H * W)
        cps = []
        for k in range(N_CHUNK):
            cp = pltpu.async_copy(
                feat_ref.at[cidx0_ref.at[k]],
                row0_ref.at[pl.ds(k * C_CHUNK, C_CHUNK)], sem)
            cp.start()
            cps.append(cp)
            cp = pltpu.async_copy(
                feat_ref.at[cidx1_ref.at[k]],
                row1_ref.at[pl.ds(k * C_CHUNK, C_CHUNK)], sem)
            cp.start()
            cps.append(cp)
        for cp in cps:
            cp.wait()
        pltpu.sync_copy(row0_ref, out_ref.at[wid])
        pltpu.sync_copy(row1_ref, out_ref.at[wid + MAX_PROPOSALS])


def _sc_gather(feat_flat, posb):
    mesh = plsc.VectorSubcoreMesh(core_axis_name="c", subcore_axis_name="s")
    kf = functools.partial(
        pl.kernel,
        out_type=jax.ShapeDtypeStruct((2 * MAX_PROPOSALS, IN_C), jnp.float32),
        mesh=mesh,
        scratch_types=[
            pltpu.VMEM((128,), jnp.int32),
            pltpu.VMEM((N_CHUNK, C_CHUNK), jnp.int32),
            pltpu.VMEM((N_CHUNK, C_CHUNK), jnp.int32),
            pltpu.VMEM((IN_C,), jnp.float32),
            pltpu.VMEM((IN_C,), jnp.float32),
            pltpu.SemaphoreType.DMA,
        ],
        name="sc_peak_gather",
    )(_sc_gather_body)
    return kf(feat_flat, posb)


def kernel(features, Wc, bc):
    wv = Wc[-1].reshape(IN_C, 1)
    b0 = bc[-1].reshape(1, 1)

    scores32, idx32, posb = pl.pallas_call(
        _center_topk_body,
        grid=(N_H_BLK,),
        in_specs=[
            pl.BlockSpec((B, IN_C, H_BLK, W), lambda j: (0, 0, j, 0)),
            pl.BlockSpec((IN_C, 1), lambda j: (0, 0)),
            pl.BlockSpec((1, 1), lambda j: (0, 0)),
        ],
        out_specs=[
            pl.BlockSpec((1, 32), lambda j: (0, 0)),
            pl.BlockSpec((1, 32), lambda j: (0, 0)),
            pl.BlockSpec((32, 128), lambda j: (0, 0)),
        ],
        out_shape=[
            jax.ShapeDtypeStruct((1, 32), jnp.float32),
            jax.ShapeDtypeStruct((1, 32), jnp.int32),
            jax.ShapeDtypeStruct((32, 128), jnp.int32),
        ],
        scratch_shapes=[
            pltpu.VMEM((H, W), jnp.float32),
        ],
    )(features, wv, b0)

    param = _sc_gather(features.reshape(-1), posb)

    scores = scores32[0, :MAX_PROPOSALS]
    pos = idx32[0, :MAX_PROPOSALS]
    y = pos // W
    x = pos % W
    instance_coord = jnp.tile(jnp.stack((y, x), axis=1), (2, 1))
    instance_imgid = jnp.concatenate([jnp.zeros((MAX_PROPOSALS,), jnp.int32),
                                      jnp.ones((MAX_PROPOSALS,), jnp.int32)])
    return (instance_coord, instance_imgid, param, scores)


# speedup vs baseline: 1.0411x; 1.0411x over previous
import functools

import jax
import jax.numpy as jnp
from jax import lax
from jax.experimental import pallas as pl
from jax.experimental.pallas import tpu as pltpu
from jax.experimental.pallas import tpu_sc as plsc

B = 2
IN_C = 768
H = 128
W = 128
MAX_PROPOSALS = 30
H_BLK = 16
N_H_BLK = H // H_BLK
L = 16
C_CHUNK = 128
N_CHUNK = IN_C // C_CHUNK


def _center_topk_body(f_ref, w_ref, b_ref, s_ref, i_ref, pb_ref, center_ref):
    j = pl.program_id(0)

    x = f_ref[...].astype(jnp.bfloat16).astype(jnp.float32)
    wv = w_ref[...].astype(jnp.bfloat16).astype(jnp.float32)[None, :, :, None]
    logits = jnp.sum(x * wv, axis=1) + b_ref[0, 0]
    p = jnp.clip(1.0 / (1.0 + jnp.exp(-logits)), 0.0001, 1.0 - 0.0001)
    center_ref[pl.ds(j * H_BLK, H_BLK), :] = (p[0] + p[1]) * 0.5

    @pl.when(j == N_H_BLK - 1)
    def _():
        center = center_ref[...]

        def win3(t, pad_val, red):
            prow = jnp.full((1, W), pad_val, dtype=t.dtype)
            pcol = jnp.full((H, 1), pad_val, dtype=t.dtype)
            up = jnp.concatenate([t[1:, :], prow], axis=0)
            dn = jnp.concatenate([prow, t[:-1, :]], axis=0)
            acc = red(red(up, t), dn)
            lf = jnp.concatenate([acc[:, 1:], pcol], axis=1)
            rt = jnp.concatenate([pcol, acc[:, :-1]], axis=1)
            return red(red(lf, acc), rt)

        pool = win3(center, 0.0, jnp.add) / 9.0
        c2 = (center + pool) * 0.5
        mx = win3(c2, -jnp.inf, jnp.maximum)
        sup = jnp.where(mx == c2, c2, 0.0)

        row = jax.lax.broadcasted_iota(jnp.int32, (H, W), 0)
        col = jax.lax.broadcasted_iota(jnp.int32, (H, W), 1)
        lin = row * W + col
        out_lane = jax.lax.broadcasted_iota(jnp.int32, (1, 32), 1)
        sub8 = jax.lax.broadcasted_iota(jnp.int32, (8, W), 0)
        col8 = jax.lax.broadcasted_iota(jnp.int32, (8, W), 1)
        base8 = sub8 * W + col8

        def body(i, carry):
            v, sc, ix = carry
            r8 = v[0:8, :]
            t8 = jnp.zeros((8, W), jnp.int32)
            for t in range(1, 16):
                vt = v[8 * t:8 * t + 8, :]
                gt = vt > r8
                r8 = jnp.where(gt, vt, r8)
                t8 = jnp.where(gt, t, t8)
            m = jnp.max(r8)
            idx = jnp.min(jnp.where(r8 == m, t8 * (8 * W) + base8,
                                    jnp.int32(2**30)))
            sc = jnp.where(out_lane == i, m, sc)
            ix = jnp.where(out_lane == i, idx, ix)
            return (jnp.where(lin == idx, -jnp.inf, v), sc, ix)

        sc0 = jnp.zeros((1, 32), jnp.float32)
        ix0 = jnp.zeros((1, 32), jnp.int32)
        _, sc, ix = jax.lax.fori_loop(0, MAX_PROPOSALS, body, (sup, sc0, ix0))
        s_ref[...] = sc
        i_ref[...] = ix
        t32 = jax.lax.transpose(jnp.broadcast_to(ix, (32, 32)), (1, 0))
        pb_ref[...] = jnp.concatenate([t32, t32, t32, t32], axis=1)


def _sc_gather_body(feat_ref, posb_ref, out_ref, posrow_ref,
                    cidx0_ref, cidx1_ref, row0_ref, row1_ref, sem):
    wid = lax.axis_index("s") * 2 + lax.axis_index("c")

    @pl.when(wid < MAX_PROPOSALS)
    def _():
        lane = lax.broadcasted_iota(jnp.int32, (L,), 0)
        pltpu.sync_copy(posb_ref.at[wid], posrow_ref)
        posv = posrow_ref[pl.ds(0, L)]
        for k in range(N_CHUNK):
            for m in range(C_CHUNK // L):
                c0 = k * C_CHUNK + m * L
                cidx0_ref[k, pl.ds(m * L, L)] = posv + (c0 + lane) * (H * W)
                cidx1_ref[k, pl.ds(m * L, L)] = (IN_C * H * W) + posv + (c0 + lane) * (H * W)
        cps = []
        for k in range(N_CHUNK):
            cp = pltpu.async_copy(
                feat_ref.at[cidx0_ref.at[k]],
                row0_ref.at[pl.ds(k * C_CHUNK, C_CHUNK)], sem)
            cp.start()
            cps.append(cp)
            cp = pltpu.async_copy(
                feat_ref.at[cidx1_ref.at[k]],
                row1_ref.at[pl.ds(k * C_CHUNK, C_CHUNK)], sem)
            cp.start()
            cps.append(cp)
        for cp in cps:
            cp.wait()
        pltpu.sync_copy(row0_ref, out_ref.at[wid])
        pltpu.sync_copy(row1_ref, out_ref.at[wid + MAX_PROPOSALS])


def _sc_gather(feat_flat, posb):
    mesh = plsc.VectorSubcoreMesh(core_axis_name="c", subcore_axis_name="s")
    kf = functools.partial(
        pl.kernel,
        out_type=jax.ShapeDtypeStruct((2 * MAX_PROPOSALS, IN_C), jnp.float32),
        mesh=mesh,
        scratch_types=[
            pltpu.VMEM((128,), jnp.int32),
            pltpu.VMEM((N_CHUNK, C_CHUNK), jnp.int32),
            pltpu.VMEM((N_CHUNK, C_CHUNK), jnp.int32),
            pltpu.VMEM((IN_C,), jnp.float32),
            pltpu.VMEM((IN_C,), jnp.float32),
            pltpu.SemaphoreType.DMA,
        ],
        name="sc_peak_gather",
    )(_sc_gather_body)
    return kf(feat_flat, posb)


def kernel(features, Wc, bc):
    wv = Wc[-1].reshape(IN_C, 1)
    b0 = bc[-1].reshape(1, 1)

    scores32, idx32, posb = pl.pallas_call(
        _center_topk_body,
        grid=(N_H_BLK,),
        in_specs=[
            pl.BlockSpec((B, IN_C, H_BLK, W), lambda j: (0, 0, j, 0)),
            pl.BlockSpec((IN_C, 1), lambda j: (0, 0)),
            pl.BlockSpec((1, 1), lambda j: (0, 0)),
        ],
        out_specs=[
            pl.BlockSpec((1, 32), lambda j: (0, 0)),
            pl.BlockSpec((1, 32), lambda j: (0, 0)),
            pl.BlockSpec((32, 128), lambda j: (0, 0)),
        ],
        out_shape=[
            jax.ShapeDtypeStruct((1, 32), jnp.float32),
            jax.ShapeDtypeStruct((1, 32), jnp.int32),
            jax.ShapeDtypeStruct((32, 128), jnp.int32),
        ],
        scratch_shapes=[
            pltpu.VMEM((H, W), jnp.float32),
        ],
    )(features, wv, b0)

    param = _sc_gather(features.reshape(-1), posb)

    scores = scores32[0, :MAX_PROPOSALS]
    pos = idx32[0, :MAX_PROPOSALS]
    y = pos // W
    x = pos % W
    instance_coord = jnp.tile(jnp.stack((y, x), axis=1), (2, 1))
    instance_imgid = jnp.concatenate([jnp.zeros((MAX_PROPOSALS,), jnp.int32),
                                      jnp.ones((MAX_PROPOSALS,), jnp.int32)])
    return (instance_coord, instance_imgid, param, scores)
